# Initial kernel scaffold; baseline (speedup 1.0000x reference)
import jax
import jax.numpy as jnp
from jax.experimental import pallas as pl


def kernel(inputs, eps):
    raise NotImplementedError("write your pallas kernel here")


# speedup vs baseline: 148.2815x; 148.2815x over previous
import jax
import jax.numpy as jnp
from jax.experimental import pallas as pl

_B_CHUNK = 8


def _qgps_body(s_ref, e0_ref, e1_ref, o_ref):
    e0 = e0_ref[...]
    e1 = e1_ref[...]
    rows = s_ref.shape[0]
    for i in range(rows // _B_CHUNK):
        s = s_ref[pl.ds(i * _B_CHUNK, _B_CHUNK), :]
        m = (s > 0)[:, None, :]
        v = jnp.where(m, e1[None], e0[None])
        d = v.shape[-1]
        while d > 1:
            d //= 2
            v = v[..., :d] * v[..., d:2 * d]
        o_ref[0, 0, pl.ds(i * _B_CHUNK, _B_CHUNK)] = jnp.sum(v[..., 0], axis=-1)


def kernel(inputs, eps):
    if inputs.ndim == 1:
        inputs = jnp.expand_dims(inputs, axis=0)
    B, L = inputs.shape
    N = eps.shape[1]
    n_blocks = 8
    rows = B // n_blocks
    out = pl.pallas_call(
        _qgps_body,
        grid=(n_blocks,),
        in_specs=[
            pl.BlockSpec((rows, L), lambda i: (i, 0)),
            pl.BlockSpec((N, L), lambda i: (0, 0)),
            pl.BlockSpec((N, L), lambda i: (0, 0)),
        ],
        out_specs=pl.BlockSpec((1, 1, rows), lambda i: (i, 0, 0)),
        out_shape=jax.ShapeDtypeStruct((n_blocks, 1, rows), jnp.float32),
    )(inputs, eps[0], eps[1])
    return out.reshape(B)


# speedup vs baseline: 1353.8977x; 9.1306x over previous
import jax
import jax.numpy as jnp
from jax.experimental import pallas as pl


def _qgps_body(s_ref, e0_ref, e1_ref, o_ref):
    sf = s_ref[...].astype(jnp.float32)
    e0 = e0_ref[...]
    e1 = e1_ref[...]
    t0 = jnp.maximum(jnp.log(jnp.abs(e0)), -1e4)
    t1 = jnp.maximum(jnp.log(jnp.abs(e1)), -1e4)
    base = jnp.sum(t0, axis=0, keepdims=True)
    logp = base + jax.lax.dot(sf, t1 - t0,
                              preferred_element_type=jnp.float32)
    n0 = (e0 < 0).astype(jnp.float32)
    n1 = (e1 < 0).astype(jnp.float32)
    cbase = jnp.sum(n0, axis=0, keepdims=True)
    negs = cbase + jax.lax.dot(sf, n1 - n0,
                               preferred_element_type=jnp.float32)
    parity = negs - 2.0 * jnp.floor(negs * 0.5)
    sign = 1.0 - 2.0 * parity
    psi = sign * jnp.exp(logp)
    o_ref[...] = jnp.sum(psi, axis=1, keepdims=True)


def kernel(inputs, eps):
    if inputs.ndim == 1:
        inputs = jnp.expand_dims(inputs, axis=0)
    B, L = inputs.shape
    N = eps.shape[1]
    e0 = eps[0].T
    e1 = eps[1].T
    out = pl.pallas_call(
        _qgps_body,
        in_specs=[
            pl.BlockSpec((B, L), lambda: (0, 0)),
            pl.BlockSpec((L, N), lambda: (0, 0)),
            pl.BlockSpec((L, N), lambda: (0, 0)),
        ],
        out_specs=pl.BlockSpec((B, 1), lambda: (0, 0)),
        out_shape=jax.ShapeDtypeStruct((B, 1), jnp.float32),
    )(inputs, e0, e1)
    return out.reshape(B)
